# speedup vs baseline: 2.4058x; 2.4058x over previous
import jax
import jax.numpy as jnp
from jax.experimental import pallas as pl

_B = 2000


def _body(z_ref, nmm_ref, gyro_ref, w_ref, out_ref):
    z = z_ref[...]
    nmm = nmm_ref[...]
    t = gyro_ref[...]
    w = w_ref[...]
    b = z.shape[0]
    lane = jax.lax.broadcasted_iota(jnp.int32, (b, 128), 1)
    gamma = jnp.sum(jnp.where(z == lane, t, 0.0), axis=1, keepdims=True)
    out_ref[...] = (gamma * nmm) * w


def kernel(Z, nuclear_magnetic_moments, gyro_table, W):
    n = Z.shape[0]
    nb = n // _B
    z2 = Z.reshape(n, 1).astype(jnp.int32)
    gyro_row = jnp.zeros((1, 128), jnp.float32).at[0, : gyro_table.shape[0]].set(
        gyro_table[:, 0]
    )
    w_row = W.reshape(1, -1)
    out = pl.pallas_call(
        _body,
        grid=(nb,),
        in_specs=[
            pl.BlockSpec((_B, 1), lambda i: (i, 0)),
            pl.BlockSpec((_B, 1), lambda i: (i, 0)),
            pl.BlockSpec((1, 128), lambda i: (0, 0)),
            pl.BlockSpec((1, 512), lambda i: (0, 0)),
        ],
        out_specs=pl.BlockSpec((_B, 512), lambda i: (i, 0)),
        out_shape=jax.ShapeDtypeStruct((n, 512), jnp.float32),
    )(z2, nuclear_magnetic_moments, gyro_row, w_row)
    return out[:, None, :]


# speedup vs baseline: 2.4605x; 1.0227x over previous
import jax
import jax.numpy as jnp
from jax.experimental import pallas as pl

_B = 4000


def _body(z_ref, nmm_ref, gyro_ref, w_ref, out_ref):
    z = z_ref[...]
    nmm = nmm_ref[...]
    t = gyro_ref[...]
    w = w_ref[...]
    b = z.shape[0]
    lane = jax.lax.broadcasted_iota(jnp.int32, (b, 128), 1)
    gamma = jnp.sum(jnp.where(z == lane, t, 0.0), axis=1, keepdims=True)
    out_ref[...] = (gamma * nmm) * w


def kernel(Z, nuclear_magnetic_moments, gyro_table, W):
    n = Z.shape[0]
    nb = n // _B
    z2 = Z.reshape(n, 1).astype(jnp.int32)
    gyro_row = jnp.zeros((1, 128), jnp.float32).at[0, : gyro_table.shape[0]].set(
        gyro_table[:, 0]
    )
    w_row = W.reshape(1, -1)
    out = pl.pallas_call(
        _body,
        grid=(nb,),
        in_specs=[
            pl.BlockSpec((_B, 1), lambda i: (i, 0)),
            pl.BlockSpec((_B, 1), lambda i: (i, 0)),
            pl.BlockSpec((1, 128), lambda i: (0, 0)),
            pl.BlockSpec((1, 512), lambda i: (0, 0)),
        ],
        out_specs=pl.BlockSpec((_B, 512), lambda i: (i, 0)),
        out_shape=jax.ShapeDtypeStruct((n, 512), jnp.float32),
    )(z2, nuclear_magnetic_moments, gyro_row, w_row)
    return out[:, None, :]


# speedup vs baseline: 4.7022x; 1.9110x over previous
import jax
import jax.numpy as jnp
from jax.experimental import pallas as pl

_B = 4000


def _body(z_ref, nmm_ref, gyro_ref, w_ref, out_ref):
    z = z_ref[...]
    nmm = nmm_ref[...]
    t = gyro_ref[...]
    w = w_ref[...]
    b = z.shape[0]
    lane = jax.lax.broadcasted_iota(jnp.int32, (b, 128), 1)
    gamma = jnp.sum(jnp.where(z == lane, t, 0.0), axis=1, keepdims=True)
    out_ref[:, 0, :] = (gamma * nmm) * w


def kernel(Z, nuclear_magnetic_moments, gyro_table, W):
    n = Z.shape[0]
    nb = n // _B
    z2 = Z.reshape(n, 1).astype(jnp.int32)
    gyro_row = jnp.zeros((1, 128), jnp.float32).at[0, : gyro_table.shape[0]].set(
        gyro_table[:, 0]
    )
    w_row = W.reshape(1, -1)
    out = pl.pallas_call(
        _body,
        grid=(nb,),
        in_specs=[
            pl.BlockSpec((_B, 1), lambda i: (i, 0)),
            pl.BlockSpec((_B, 1), lambda i: (i, 0)),
            pl.BlockSpec((1, 128), lambda i: (0, 0)),
            pl.BlockSpec((1, 512), lambda i: (0, 0)),
        ],
        out_specs=pl.BlockSpec((_B, 1, 512), lambda i: (i, 0, 0)),
        out_shape=jax.ShapeDtypeStruct((n, 1, 512), jnp.float32),
    )(z2, nuclear_magnetic_moments, gyro_row, w_row)
    return out


# speedup vs baseline: 4.8203x; 1.0251x over previous
import jax
import jax.numpy as jnp
from jax.experimental import pallas as pl

_B = 4000


def _body(z_ref, nmm_ref, gyro_ref, w_ref, out_ref):
    z = z_ref[...]
    nmm = nmm_ref[...]
    tcol = gyro_ref[...]
    w = w_ref[...]
    b = z.shape[0]
    lane = jax.lax.broadcasted_iota(jnp.int32, (b, 128), 1)
    a = jnp.where(z == lane, nmm, 0.0)
    m = tcol * w
    out_ref[...] = jnp.dot(a, m, preferred_element_type=jnp.float32)


def kernel(Z, nuclear_magnetic_moments, gyro_table, W):
    n = Z.shape[0]
    nb = n // _B
    z2 = Z.reshape(n, 1).astype(jnp.int32)
    gyro_col = jnp.zeros((128, 1), jnp.float32).at[: gyro_table.shape[0]].set(
        gyro_table
    )
    w_row = W.reshape(1, -1)
    out = pl.pallas_call(
        _body,
        grid=(nb,),
        in_specs=[
            pl.BlockSpec((_B, 1), lambda i: (i, 0)),
            pl.BlockSpec((_B, 1), lambda i: (i, 0)),
            pl.BlockSpec((128, 1), lambda i: (0, 0)),
            pl.BlockSpec((1, 512), lambda i: (0, 0)),
        ],
        out_specs=pl.BlockSpec((_B, None, 512), lambda i: (i, 0, 0)),
        out_shape=jax.ShapeDtypeStruct((n, 1, 512), jnp.float32),
    )(z2, nuclear_magnetic_moments, gyro_col, w_row)
    return out


# speedup vs baseline: 7.7285x; 1.6033x over previous
import functools

import jax
import jax.numpy as jnp
from jax import lax
from jax.experimental import pallas as pl
from jax.experimental.pallas import tpu as pltpu
from jax.experimental.pallas import tpu_sc as plsc

_NW = 32
_PW = 3200
_RC = 80
_NQ = _PW // _RC
_D = 512
_L = 16


def _make_sc_kernel(n):
    mesh = plsc.VectorSubcoreMesh(core_axis_name="c", subcore_axis_name="s")

    @functools.partial(
        pl.kernel,
        mesh=mesh,
        compiler_params=pltpu.CompilerParams(needs_layout_passes=False),
        out_type=jax.ShapeDtypeStruct((n, 1, _D), jnp.float32),
        scratch_types=[
            pltpu.VMEM((_PW,), jnp.int32),
            pltpu.VMEM((_PW,), jnp.float32),
            pltpu.VMEM((128,), jnp.float32),
            pltpu.VMEM((_D,), jnp.float32),
            pltpu.VMEM((_PW,), jnp.float32),
            pltpu.VMEM((_RC, 1, _D), jnp.float32),
            pltpu.VMEM((_RC, 1, _D), jnp.float32),
            pltpu.SemaphoreType.DMA,
            pltpu.SemaphoreType.DMA,
        ],
    )
    def sc_k(z_hbm, nmm_hbm, gyro_hbm, w_hbm, out_hbm,
             z_v, nmm_v, t_v, w_v, s_v, buf0, buf1, sem0, sem1):
        wid = lax.axis_index("s") * 2 + lax.axis_index("c")
        base = wid * _PW
        pltpu.sync_copy(z_hbm.at[pl.ds(base, _PW)], z_v)
        pltpu.sync_copy(nmm_hbm.at[pl.ds(base, _PW)], nmm_v)
        pltpu.sync_copy(gyro_hbm, t_v)
        pltpu.sync_copy(w_hbm, w_v)

        def s_body(i, carry):
            sl = pl.ds(i * _L, _L)
            g = plsc.load_gather(t_v, [z_v[sl]])
            s_v[sl] = g * nmm_v[sl]
            return carry

        lax.fori_loop(0, _PW // _L, s_body, 0)

        wl = [w_v[pl.ds(l * _L, _L)] for l in range(_D // _L)]

        def chunk(q, buf, sem):
            row0 = base + q * _RC

            @pl.when(row0 < n)
            def _():
                @pl.when(q >= 2)
                def _():
                    pltpu.make_async_copy(
                        out_hbm.at[pl.ds(0, _RC)], buf, sem
                    ).wait()

                def row_body(j, carry):
                    a = q * _RC + j
                    sj = plsc.load_gather(s_v, [jnp.full((_L,), 0, jnp.int32) + a])
                    for l in range(_D // _L):
                        buf[j, 0, pl.ds(l * _L, _L)] = sj * wl[l]
                    return carry

                lax.fori_loop(0, _RC, row_body, 0)
                pltpu.async_copy(buf, out_hbm.at[pl.ds(row0, _RC)], sem)

        def q_body(k, carry):
            chunk(k * 2, buf0, sem0)
            chunk(k * 2 + 1, buf1, sem1)
            return carry

        lax.fori_loop(0, _NQ // 2, q_body, 0)
        pltpu.make_async_copy(out_hbm.at[pl.ds(0, _RC)], buf0, sem0).wait()
        pltpu.make_async_copy(out_hbm.at[pl.ds(0, _RC)], buf1, sem1).wait()

    return sc_k


def kernel(Z, nuclear_magnetic_moments, gyro_table, W):
    n = Z.shape[0]
    npad = _NW * _PW
    z_p = jnp.zeros((npad,), jnp.int32).at[:n].set(Z.astype(jnp.int32))
    nmm_p = jnp.zeros((npad,), jnp.float32).at[:n].set(
        nuclear_magnetic_moments[:, 0]
    )
    gyro_pad = jnp.zeros((128,), jnp.float32).at[: gyro_table.shape[0]].set(
        gyro_table[:, 0]
    )
    w_flat = W[:, 0]
    return _make_sc_kernel(n)(z_p, nmm_p, gyro_pad, w_flat)
